# baseline (speedup 1.0000x reference)
import jax
import jax.numpy as jnp
from jax.experimental import pallas as pl
from jax.experimental.pallas import tpu as pltpu

_DIM = 192
_HEADS = 4
_CH = 48
_H = 224
_W = 224
_HW = _H * _W
_NT = 1024
_GN = _HW // _NT
_WT = 14
_NWIN = 784
_GW = _NWIN // _WT
_CB = 16
_SP_KS = (32, 42, 48, 51)
_CH_KS = (24, 32, 36, 38)


def _qkv_mm_kernel(w_ref, x_ref, o_ref):
    o_ref[...] = jax.lax.dot(
        w_ref[...], x_ref[...],
        preferred_element_type=jnp.float32).astype(jnp.bfloat16)


def _dw_kernel(x_ref, w_ref, o_ref, nsq_ref):
    x = x_ref[...]
    w = w_ref[...]
    xp = jnp.pad(x, ((0, 0), (1, 1), (1, 1)))
    acc = w[:, 0:1, None] * xp[:, 0:_H, 0:_W]
    for i in range(1, 9):
        dy, dx = divmod(i, 3)
        acc = acc + w[:, i:i + 1, None] * xp[:, dy:dy + _H, dx:dx + _W]
    o_ref[...] = acc
    nsq_ref[...] = jnp.sum(acc * acc, axis=(1, 2), keepdims=True)[:, :, 0]


def _gram_kernel(q_ref, k_ref, iq_ref, ik_ref, s_ref):
    j = pl.program_id(0)

    @pl.when(j == 0)
    def _init():
        s_ref[...] = jnp.zeros_like(s_ref)

    qn = (q_ref[...] * iq_ref[...]).astype(jnp.bfloat16)
    kn = (k_ref[...] * ik_ref[...]).astype(jnp.bfloat16)
    for h in range(_HEADS):
        s_ref[h] += jax.lax.dot_general(
            qn[h * _CH:(h + 1) * _CH, :], kn[h * _CH:(h + 1) * _CH, :],
            (((1,), (1,)), ((), ())),
            preferred_element_type=jnp.float32)


def _combine_weights(a, e, ks, alphas):
    n = a.shape[-1]
    chunk = 8
    rks = []
    for j0 in range(0, n, chunk):
        ac = a[..., :, j0:j0 + chunk]
        gt = (a[..., :, None, :] > ac[..., :, :, None])
        rks.append(jnp.sum(gt.astype(jnp.float32), axis=-1))
    ranks = jnp.concatenate(rks, axis=-1)
    wsum = jnp.zeros_like(a)
    for kk, al in zip(ks, alphas):
        mask = (ranks < kk).astype(jnp.float32)
        den = jnp.sum(e * mask, axis=-1, keepdims=True)
        wsum = wsum + mask * (al / den)
    return e * wsum


def _spatial_kernel(t_ref, al_ref, ws_ref, o_ref):
    alphas = [al_ref[0], al_ref[1], al_ref[2], al_ref[3]]
    for h in range(_HEADS):
        q = ws_ref[0, :, h]
        k = ws_ref[1, :, h]
        v = ws_ref[2, :, h]
        t = t_ref[h]

        qn = (q / jnp.maximum(
            jnp.sqrt(jnp.sum(q * q, axis=-1, keepdims=True)),
            1e-12)).astype(jnp.bfloat16)
        kn = (k / jnp.maximum(
            jnp.sqrt(jnp.sum(k * k, axis=-1, keepdims=True)),
            1e-12)).astype(jnp.bfloat16)
        a = jax.lax.dot_general(
            qn, kn, (((2,), (2,)), ((0,), (0,))),
            preferred_element_type=jnp.float32) * t
        e = jnp.exp(a - jnp.max(a, axis=-1, keepdims=True))
        attn = _combine_weights(a, e, _SP_KS, alphas)
        o_ref[:, h] = jax.lax.dot_general(
            attn.astype(jnp.bfloat16), v.astype(jnp.bfloat16),
            (((2,), (1,)), ((0,), (0,))),
            preferred_element_type=jnp.float32)


def _channel_kernel(t_ref, al_ref, s_ref, ca_ref):
    alphas = [al_ref[0], al_ref[1], al_ref[2], al_ref[3]]
    for h in range(_HEADS):
        a = s_ref[h] * t_ref[h]
        e = jnp.exp(a - jnp.max(a, axis=-1, keepdims=True))
        ca_ref[h] = _combine_weights(a, e, _CH_KS, alphas)


def _out_kernel(ca_ref, p_ref, v_ref, sp_ref, o_ref):
    v = v_ref[...].astype(jnp.bfloat16)
    rows = []
    for h in range(_HEADS):
        rows.append(jax.lax.dot(
            ca_ref[h].astype(jnp.bfloat16), v[h * _CH:(h + 1) * _CH, :],
            preferred_element_type=jnp.float32))
    comb = jnp.concatenate(rows, axis=0) + sp_ref[...]
    o_ref[...] = jax.lax.dot(
        p_ref[...], comb.astype(jnp.bfloat16),
        preferred_element_type=jnp.float32)


def kernel(x, qkv_w, dw_w, proj_w, temperature, attn1, attn2, attn3, attn4):
    x2d = x.reshape(_DIM, _HW).astype(jnp.bfloat16)
    w1 = qkv_w.reshape(_DIM * 3, _DIM).astype(jnp.bfloat16)
    wd = dw_w.reshape(_DIM * 3, 9)
    pw = proj_w.reshape(_DIM, _DIM).astype(jnp.bfloat16)
    tvec = temperature.reshape(_HEADS)
    avec = jnp.concatenate([attn1, attn2, attn3, attn4])

    qkv2d = pl.pallas_call(
        _qkv_mm_kernel,
        grid=(_GN,),
        in_specs=[
            pl.BlockSpec((_DIM * 3, _DIM), lambda j: (0, 0)),
            pl.BlockSpec((_DIM, _NT), lambda j: (0, j)),
        ],
        out_specs=pl.BlockSpec((_DIM * 3, _NT), lambda j: (0, j)),
        out_shape=jax.ShapeDtypeStruct((_DIM * 3, _HW), jnp.bfloat16),
    )(w1, x2d)

    dw, nsq = pl.pallas_call(
        _dw_kernel,
        grid=(_DIM * 3 // _CB,),
        in_specs=[
            pl.BlockSpec((_CB, _H, _W), lambda j: (j, 0, 0)),
            pl.BlockSpec((_CB, 9), lambda j: (j, 0)),
        ],
        out_specs=[
            pl.BlockSpec((_CB, _H, _W), lambda j: (j, 0, 0)),
            pl.BlockSpec((_CB, 1), lambda j: (j, 0)),
        ],
        out_shape=[
            jax.ShapeDtypeStruct((_DIM * 3, _H, _W), jnp.float32),
            jax.ShapeDtypeStruct((_DIM * 3, 1), jnp.float32),
        ],
    )(qkv2d.reshape(_DIM * 3, _H, _W), wd)

    dw2d = dw.reshape(_DIM * 3, _HW)
    inv_n = 1.0 / jnp.maximum(jnp.sqrt(nsq), 1e-12)

    s_acc = pl.pallas_call(
        _gram_kernel,
        grid=(_GN,),
        in_specs=[
            pl.BlockSpec((_DIM, _NT), lambda j: (0, j)),
            pl.BlockSpec((_DIM, _NT), lambda j: (1, j)),
            pl.BlockSpec((_DIM, 1), lambda j: (0, 0)),
            pl.BlockSpec((_DIM, 1), lambda j: (1, 0)),
        ],
        out_specs=pl.BlockSpec((_HEADS, _CH, _CH), lambda j: (0, 0, 0)),
        out_shape=jax.ShapeDtypeStruct((_HEADS, _CH, _CH), jnp.float32),
    )(dw2d, dw2d, inv_n, inv_n)

    ca = pl.pallas_call(
        _channel_kernel,
        in_specs=[
            pl.BlockSpec(memory_space=pltpu.SMEM),
            pl.BlockSpec(memory_space=pltpu.SMEM),
            pl.BlockSpec((_HEADS, _CH, _CH), lambda: (0, 0, 0)),
        ],
        out_specs=pl.BlockSpec((_HEADS, _CH, _CH), lambda: (0, 0, 0)),
        out_shape=jax.ShapeDtypeStruct((_HEADS, _CH, _CH), jnp.float32),
    )(tvec, avec, s_acc)

    ws = jnp.transpose(
        dw.reshape(3, _HEADS, _CH, 28, 8, 28, 8),
        (0, 1, 3, 5, 4, 6, 2)).reshape(3, _NWIN, _HEADS, 64, _CH)

    so_w = pl.pallas_call(
        _spatial_kernel,
        grid=(_GW,),
        in_specs=[
            pl.BlockSpec(memory_space=pltpu.SMEM),
            pl.BlockSpec(memory_space=pltpu.SMEM),
            pl.BlockSpec((3, _WT, _HEADS, 64, _CH),
                         lambda j: (0, j, 0, 0, 0)),
        ],
        out_specs=pl.BlockSpec((_WT, _HEADS, 64, _CH), lambda j: (j, 0, 0, 0)),
        out_shape=jax.ShapeDtypeStruct((_NWIN, _HEADS, 64, _CH), jnp.float32),
    )(tvec, avec, ws)

    sp2d = jnp.transpose(so_w.reshape(_HEADS, _H, _W, _CH),
                         (0, 3, 1, 2)).reshape(_DIM, _HW)

    out2d = pl.pallas_call(
        _out_kernel,
        grid=(_GN,),
        in_specs=[
            pl.BlockSpec((_HEADS, _CH, _CH), lambda j: (0, 0, 0)),
            pl.BlockSpec((_DIM, _DIM), lambda j: (0, 0)),
            pl.BlockSpec((_DIM, _NT), lambda j: (2, j)),
            pl.BlockSpec((_DIM, _NT), lambda j: (0, j)),
        ],
        out_specs=pl.BlockSpec((_DIM, _NT), lambda j: (0, j)),
        out_shape=jax.ShapeDtypeStruct((_DIM, _HW), jnp.float32),
    )(ca, pw, dw2d, sp2d)

    return out2d.reshape(1, _DIM, _H, _W)


# speedup vs baseline: 1.1071x; 1.1071x over previous
import jax
import jax.numpy as jnp
from jax.experimental import pallas as pl
from jax.experimental.pallas import tpu as pltpu

_DIM = 192
_HEADS = 4
_CH = 48
_H = 224
_W = 224
_HW = _H * _W
_NT = 1024
_GN = _HW // _NT
_WT = 7
_NWIN = 784
_GW = _NWIN // _WT
_CB = 16
_SP_KS = (32, 42, 48, 51)
_CH_KS = (24, 32, 36, 38)


def _qkv_mm_kernel(w_ref, x_ref, o_ref):
    o_ref[...] = jax.lax.dot(
        w_ref[...], x_ref[...],
        preferred_element_type=jnp.float32).astype(jnp.bfloat16)


def _dw_kernel(x_ref, w_ref, o_ref, nsq_ref):
    x = x_ref[...]
    w = w_ref[...]
    xp = jnp.pad(x, ((0, 0), (1, 1), (1, 1)))
    acc = w[:, 0:1, None] * xp[:, 0:_H, 0:_W]
    for i in range(1, 9):
        dy, dx = divmod(i, 3)
        acc = acc + w[:, i:i + 1, None] * xp[:, dy:dy + _H, dx:dx + _W]
    o_ref[...] = acc
    nsq_ref[...] = jnp.sum(acc * acc, axis=(1, 2), keepdims=True)[:, :, 0]


def _gram_kernel(q_ref, k_ref, iq_ref, ik_ref, s_ref):
    j = pl.program_id(0)

    @pl.when(j == 0)
    def _init():
        s_ref[...] = jnp.zeros_like(s_ref)

    qn = (q_ref[...] * iq_ref[...]).astype(jnp.bfloat16)
    kn = (k_ref[...] * ik_ref[...]).astype(jnp.bfloat16)
    for h in range(_HEADS):
        s_ref[h] += jax.lax.dot_general(
            qn[h * _CH:(h + 1) * _CH, :], kn[h * _CH:(h + 1) * _CH, :],
            (((1,), (1,)), ((), ())),
            preferred_element_type=jnp.float32)


def _combine_weights(a, e, ks, alphas):
    n = a.shape[-1]
    chunk = 32
    rks = []
    for j0 in range(0, n, chunk):
        ac = a[..., :, j0:j0 + chunk]
        gt = (a[..., :, None, :] > ac[..., :, :, None])
        rks.append(jnp.sum(gt.astype(jnp.float32), axis=-1))
    ranks = jnp.concatenate(rks, axis=-1)
    wsum = jnp.zeros_like(a)
    for kk, al in zip(ks, alphas):
        mask = (ranks < kk).astype(jnp.float32)
        den = jnp.sum(e * mask, axis=-1, keepdims=True)
        wsum = wsum + mask * (al / den)
    return e * wsum


def _spatial_kernel(t_ref, al_ref, ws_ref, o_ref):
    alphas = [al_ref[0], al_ref[1], al_ref[2], al_ref[3]]
    for h in range(_HEADS):
        q = ws_ref[0, :, h]
        k = ws_ref[1, :, h]
        v = ws_ref[2, :, h]
        t = t_ref[h]

        qn = (q / jnp.maximum(
            jnp.sqrt(jnp.sum(q * q, axis=-1, keepdims=True)),
            1e-12)).astype(jnp.bfloat16)
        kn = (k / jnp.maximum(
            jnp.sqrt(jnp.sum(k * k, axis=-1, keepdims=True)),
            1e-12)).astype(jnp.bfloat16)
        a = jax.lax.dot_general(
            qn, kn, (((2,), (2,)), ((0,), (0,))),
            preferred_element_type=jnp.float32) * t
        e = jnp.exp(a - jnp.max(a, axis=-1, keepdims=True))
        attn = _combine_weights(a, e, _SP_KS, alphas)
        o_ref[:, h] = jax.lax.dot_general(
            attn.astype(jnp.bfloat16), v.astype(jnp.bfloat16),
            (((2,), (1,)), ((0,), (0,))),
            preferred_element_type=jnp.float32)


def _channel_kernel(t_ref, al_ref, s_ref, ca_ref):
    alphas = [al_ref[0], al_ref[1], al_ref[2], al_ref[3]]
    for h in range(_HEADS):
        a = s_ref[h] * t_ref[h]
        e = jnp.exp(a - jnp.max(a, axis=-1, keepdims=True))
        ca_ref[h] = _combine_weights(a, e, _CH_KS, alphas)


def _out_kernel(ca_ref, p_ref, v_ref, sp_ref, o_ref):
    v = v_ref[...].astype(jnp.bfloat16)
    rows = []
    for h in range(_HEADS):
        rows.append(jax.lax.dot(
            ca_ref[h].astype(jnp.bfloat16), v[h * _CH:(h + 1) * _CH, :],
            preferred_element_type=jnp.float32))
    comb = jnp.concatenate(rows, axis=0) + sp_ref[...]
    o_ref[...] = jax.lax.dot(
        p_ref[...], comb.astype(jnp.bfloat16),
        preferred_element_type=jnp.float32)


def kernel(x, qkv_w, dw_w, proj_w, temperature, attn1, attn2, attn3, attn4):
    x2d = x.reshape(_DIM, _HW).astype(jnp.bfloat16)
    w1 = qkv_w.reshape(_DIM * 3, _DIM).astype(jnp.bfloat16)
    wd = dw_w.reshape(_DIM * 3, 9)
    pw = proj_w.reshape(_DIM, _DIM).astype(jnp.bfloat16)
    tvec = temperature.reshape(_HEADS)
    avec = jnp.concatenate([attn1, attn2, attn3, attn4])

    qkv2d = pl.pallas_call(
        _qkv_mm_kernel,
        grid=(_GN,),
        in_specs=[
            pl.BlockSpec((_DIM * 3, _DIM), lambda j: (0, 0)),
            pl.BlockSpec((_DIM, _NT), lambda j: (0, j)),
        ],
        out_specs=pl.BlockSpec((_DIM * 3, _NT), lambda j: (0, j)),
        out_shape=jax.ShapeDtypeStruct((_DIM * 3, _HW), jnp.bfloat16),
    )(w1, x2d)

    dw, nsq = pl.pallas_call(
        _dw_kernel,
        grid=(_DIM * 3 // _CB,),
        in_specs=[
            pl.BlockSpec((_CB, _H, _W), lambda j: (j, 0, 0)),
            pl.BlockSpec((_CB, 9), lambda j: (j, 0)),
        ],
        out_specs=[
            pl.BlockSpec((_CB, _H, _W), lambda j: (j, 0, 0)),
            pl.BlockSpec((_CB, 1), lambda j: (j, 0)),
        ],
        out_shape=[
            jax.ShapeDtypeStruct((_DIM * 3, _H, _W), jnp.float32),
            jax.ShapeDtypeStruct((_DIM * 3, 1), jnp.float32),
        ],
    )(qkv2d.reshape(_DIM * 3, _H, _W), wd)

    dw2d = dw.reshape(_DIM * 3, _HW)
    inv_n = 1.0 / jnp.maximum(jnp.sqrt(nsq), 1e-12)

    s_acc = pl.pallas_call(
        _gram_kernel,
        grid=(_GN,),
        in_specs=[
            pl.BlockSpec((_DIM, _NT), lambda j: (0, j)),
            pl.BlockSpec((_DIM, _NT), lambda j: (1, j)),
            pl.BlockSpec((_DIM, 1), lambda j: (0, 0)),
            pl.BlockSpec((_DIM, 1), lambda j: (1, 0)),
        ],
        out_specs=pl.BlockSpec((_HEADS, _CH, _CH), lambda j: (0, 0, 0)),
        out_shape=jax.ShapeDtypeStruct((_HEADS, _CH, _CH), jnp.float32),
    )(dw2d, dw2d, inv_n, inv_n)

    ca = pl.pallas_call(
        _channel_kernel,
        in_specs=[
            pl.BlockSpec(memory_space=pltpu.SMEM),
            pl.BlockSpec(memory_space=pltpu.SMEM),
            pl.BlockSpec((_HEADS, _CH, _CH), lambda: (0, 0, 0)),
        ],
        out_specs=pl.BlockSpec((_HEADS, _CH, _CH), lambda: (0, 0, 0)),
        out_shape=jax.ShapeDtypeStruct((_HEADS, _CH, _CH), jnp.float32),
    )(tvec, avec, s_acc)

    ws = jnp.transpose(
        dw.reshape(3, _HEADS, _CH, 28, 8, 28, 8),
        (0, 1, 3, 5, 4, 6, 2)).reshape(3, _NWIN, _HEADS, 64, _CH)

    so_w = pl.pallas_call(
        _spatial_kernel,
        grid=(_GW,),
        in_specs=[
            pl.BlockSpec(memory_space=pltpu.SMEM),
            pl.BlockSpec(memory_space=pltpu.SMEM),
            pl.BlockSpec((3, _WT, _HEADS, 64, _CH),
                         lambda j: (0, j, 0, 0, 0)),
        ],
        out_specs=pl.BlockSpec((_WT, _HEADS, 64, _CH), lambda j: (j, 0, 0, 0)),
        out_shape=jax.ShapeDtypeStruct((_NWIN, _HEADS, 64, _CH), jnp.float32),
    )(tvec, avec, ws)

    sp2d = jnp.transpose(so_w.reshape(_HEADS, _H, _W, _CH),
                         (0, 3, 1, 2)).reshape(_DIM, _HW)

    out2d = pl.pallas_call(
        _out_kernel,
        grid=(_GN,),
        in_specs=[
            pl.BlockSpec((_HEADS, _CH, _CH), lambda j: (0, 0, 0)),
            pl.BlockSpec((_DIM, _DIM), lambda j: (0, 0)),
            pl.BlockSpec((_DIM, _NT), lambda j: (2, j)),
            pl.BlockSpec((_DIM, _NT), lambda j: (0, j)),
        ],
        out_specs=pl.BlockSpec((_DIM, _NT), lambda j: (0, j)),
        out_shape=jax.ShapeDtypeStruct((_DIM, _HW), jnp.float32),
    )(ca, pw, dw2d, sp2d)

    return out2d.reshape(1, _DIM, _H, _W)
